# baseline (speedup 1.0000x reference)
import jax
import jax.numpy as jnp
from jax.experimental import pallas as pl

_N = 10000
_G = 64


def _gat_layer(h, src, dst, W, a_src, a_dst, b):
    hp = h @ W
    alpha_src = (hp * a_src).sum(-1)
    alpha_dst = (hp * a_dst).sum(-1)
    e = jax.nn.leaky_relu(alpha_src[src] + alpha_dst[dst], negative_slope=0.2)
    m = jax.ops.segment_max(e, dst, num_segments=_N)
    ex = jnp.exp(e - m[dst])
    denom = jax.ops.segment_sum(ex, dst, num_segments=_N)
    alpha = ex / (denom[dst] + 1e-16)
    out = jax.ops.segment_sum(hp[src] * alpha[:, None], dst, num_segments=_N)
    return out + b


def _pool(h, batch):
    s = jax.ops.segment_sum(h, batch, num_segments=_G)
    cnt = jax.ops.segment_sum(jnp.ones((h.shape[0],), h.dtype), batch, num_segments=_G)
    return s / jnp.maximum(cnt, 1.0)[:, None]


def kernel(x, edge_index, edge_weight, batch, W1, a_src1, a_dst1, b1, W2, a_src2, a_dst2, b2):
    loops = jnp.arange(_N, dtype=edge_index.dtype)
    src = jnp.concatenate([edge_index[0], loops])
    dst = jnp.concatenate([edge_index[1], loops])
    h1 = jax.nn.relu(_gat_layer(x, src, dst, W1, a_src1, a_dst1, b1))
    emb1 = _pool(h1, batch)
    h2 = _gat_layer(h1, src, dst, W2, a_src2, a_dst2, b2)
    emb2 = _pool(h2, batch)
    return (emb1, emb2)


# speedup vs baseline: 1.4395x; 1.4395x over previous
import functools

import jax
import jax.numpy as jnp
from jax import lax
from jax.experimental import pallas as pl
from jax.experimental.pallas import tpu as pltpu
from jax.experimental.pallas import tpu_sc as plsc

N = 10000
E = 320000
D = 128
G = 64

NP = 10240
HALF = NP // 2
ET = E + N
NCH = 168
EPT = NCH * 128
ET_PAD = 16 * EPT

QROWS = HALF // 4
DTR = HALF

BF = 1024
GRIDF = NP // BF


def _proj_body(x_ref, w_ref, as_ref, ad_ref, hp_ref, s_ref, d_ref, m_ref,
               mm_ref, *, grid):
    i = pl.program_id(0)
    hp = jnp.dot(x_ref[...], w_ref[...], preferred_element_type=jnp.float32)
    hp_ref[...] = hp
    s = jnp.sum(hp * as_ref[...], axis=1)
    d = jnp.sum(hp * ad_ref[...], axis=1)
    s_ref[...] = s.reshape(BF // 128, 128)
    d_ref[...] = d.reshape(BF // 128, 128)
    valid = i * BF + lax.iota(jnp.int32, BF) < N
    ms = jnp.max(jnp.where(valid, s, -3e38))
    md = jnp.max(jnp.where(valid, d, -3e38))

    @pl.when(i == 0)
    def _():
        mm_ref[0] = ms
        mm_ref[1] = md

    @pl.when(i > 0)
    def _():
        mm_ref[0] = jnp.maximum(mm_ref[0], ms)
        mm_ref[1] = jnp.maximum(mm_ref[1], md)

    @pl.when(i == grid - 1)
    def _():
        z = mm_ref[0] + mm_ref[1]
        m_ref[0, 0] = jnp.where(z >= 0, z, 0.2 * z)


def _project(h, W, a_s, a_d):
    grid = GRIDF
    body = functools.partial(_proj_body, grid=grid)
    outs = pl.pallas_call(
        body,
        grid=(grid,),
        in_specs=[
            pl.BlockSpec((BF, D), lambda i: (i, 0)),
            pl.BlockSpec((D, D), lambda i: (0, 0)),
            pl.BlockSpec((1, D), lambda i: (0, 0)),
            pl.BlockSpec((1, D), lambda i: (0, 0)),
        ],
        out_specs=[
            pl.BlockSpec((BF, D), lambda i: (i, 0)),
            pl.BlockSpec((BF // 128, 128), lambda i: (i, 0)),
            pl.BlockSpec((BF // 128, 128), lambda i: (i, 0)),
            pl.BlockSpec((1, 1), lambda i: (0, 0), memory_space=pltpu.SMEM),
        ],
        out_shape=[
            jax.ShapeDtypeStruct((NP, D), jnp.float32),
            jax.ShapeDtypeStruct((NP // 128, 128), jnp.float32),
            jax.ShapeDtypeStruct((NP // 128, 128), jnp.float32),
            jax.ShapeDtypeStruct((1, 1), jnp.float32),
        ],
        scratch_shapes=[pltpu.SMEM((2,), jnp.float32)],
    )(h, W, a_s.reshape(1, D), a_d.reshape(1, D))
    hp, s, d, m = outs
    return hp, s, d, m


def _prep_body(e_ref, src_ref, dst_ref):
    tail_rows = 16 * NCH - E // 128
    fi = (lax.broadcasted_iota(jnp.int32, (tail_rows, 128), 0) * 128
          + lax.broadcasted_iota(jnp.int32, (tail_rows, 128), 1))
    tail = jnp.where(fi < N, fi, 0)
    for k, out_ref in ((0, src_ref), (1, dst_ref)):
        cat = jnp.concatenate([e_ref[k], tail], axis=0)
        out_ref[...] = cat.reshape(16, NCH, 128)


def _prep(e3):
    return pl.pallas_call(
        _prep_body,
        out_shape=[
            jax.ShapeDtypeStruct((16, NCH, 128), jnp.int32),
            jax.ShapeDtypeStruct((16, NCH, 128), jnp.int32),
        ],
    )(e3)


def _edge_body(hp_hbm, src_hbm, dst_hbm, asrc_hbm, adst_hbm, m_hbm,
               outp_hbm, denp_hbm,
               asrc_v, adst_v, src_v, dst_v, ex_v, rows_g, rows_s, idq_v,
               idd_v, m_v, zden_v, sem,
               out_sp, den_sp):
    cid = lax.axis_index("c")
    sid = lax.axis_index("s")
    gbase = cid * HALF

    pltpu.sync_copy(asrc_hbm, asrc_v)
    pltpu.sync_copy(adst_hbm, adst_v)
    pltpu.sync_copy(src_hbm.at[sid], src_v)
    pltpu.sync_copy(dst_hbm.at[sid], dst_v)
    pltpu.sync_copy(m_hbm, m_v)
    mvec = m_v[...]
    zcol = jnp.zeros((16,), jnp.int32)

    zero16 = jnp.zeros((16,), jnp.float32)

    def _zr(r, carry):
        for j in range(D // 16):
            rows_g[r, pl.ds(j * 16, 16)] = zero16
        return carry

    lax.fori_loop(0, 128, _zr, 0)

    def _zd(k, carry):
        zden_v[pl.ds(k * 16, 16)] = zero16
        return carry

    lax.fori_loop(0, (HALF // 16 + 16) // 16, _zd, 0)

    pltpu.sync_copy(zden_v, den_sp.at[pl.ds(sid * (HALF // 16 + 16),
                                            HALF // 16 + 16)])

    plsc.subcore_barrier()

    ebase = sid * EPT

    def _phase_a(c, carry):
        for j in range(8):
            sl = pl.ds(j * 16, 16)
            si = src_v[c, sl]
            di = dst_v[c, sl]
            s = plsc.load_gather(asrc_v, [lax.shift_right_logical(si, 7),
                                          lax.bitwise_and(si, 127)])
            d = plsc.load_gather(adst_v, [lax.shift_right_logical(di, 7),
                                          lax.bitwise_and(di, 127)])
            z = s + d
            e = jnp.where(z >= 0, z, 0.2 * z)
            ex = jnp.exp(e - mvec)
            gi = ebase + c * 128 + j * 16 + lax.iota(jnp.int32, 16)
            ex = jnp.where(gi < ET, ex, 0.0)
            ex_v[c, sl] = ex
            ld = di - gbase
            inh = (ld >= 0) & (ld < HALF)
            idd_v[sl] = jnp.where(inh, ld, DTR)
        pltpu.sync_copy(ex_v.at[c], den_sp.at[idd_v], add=True)
        return carry

    lax.fori_loop(0, NCH, _phase_a, 0)

    spt = QROWS // 16

    for q in range(HALF // QROWS):
        qb = q * QROWS

        lax.fori_loop(0, spt, _zr, 0)
        pltpu.sync_copy(rows_g.at[pl.ds(0, spt)],
                        out_sp.at[pl.ds(sid * spt, spt)])
        plsc.subcore_barrier()

        def _phase_b(c, carry):
            pltpu.async_copy(hp_hbm.at[src_v.at[c]], rows_g, sem).wait()

            for j in range(8):
                sl = pl.ds(j * 16, 16)
                ld = dst_v[c, sl] - gbase
                idq_v[sl] = jnp.clip(ld - qb, 0, QROWS - 1)

            def _scale(r, carry2):
                rbc = jnp.full((16,), r, jnp.int32)
                w = plsc.load_gather(ex_v.at[c], [rbc])
                dv = plsc.load_gather(dst_v.at[c], [rbc]) - gbase
                wr = jnp.where((dv >= qb) & (dv < qb + QROWS), w, 0.0)
                for j in range(D // 16):
                    sl = pl.ds(j * 16, 16)
                    rows_s[r, sl] = rows_g[r, sl] * wr
                return carry2

            lax.fori_loop(0, 128, _scale, 0)
            pltpu.sync_copy(rows_s, out_sp.at[idq_v], add=True)
            return carry

        lax.fori_loop(0, NCH, _phase_b, 0)
        plsc.subcore_barrier()

        pltpu.sync_copy(out_sp.at[pl.ds(sid * spt, spt)],
                        outp_hbm.at[pl.ds(gbase + qb + sid * spt, spt)])
        plsc.subcore_barrier()

    dpt = HALF // 16
    pltpu.sync_copy(den_sp.at[pl.ds(sid * dpt, dpt)],
                    denp_hbm.at[pl.ds(gbase + sid * dpt, dpt)])


def _make_edge_agg(interpret=False):
    return pl.kernel(
        _edge_body,
        out_type=[
            jax.ShapeDtypeStruct((NP, D), jnp.float32),
            jax.ShapeDtypeStruct((NP,), jnp.float32),
        ],
        mesh=plsc.VectorSubcoreMesh(core_axis_name="c", subcore_axis_name="s"),
        compiler_params=pltpu.CompilerParams(needs_layout_passes=False,
                                             use_tc_tiling_on_sc=False),
        interpret=interpret,
        scratch_types=[
            pltpu.VMEM((NP // 128, 128), jnp.float32),
            pltpu.VMEM((NP // 128, 128), jnp.float32),
            pltpu.VMEM((NCH, 128), jnp.int32),
            pltpu.VMEM((NCH, 128), jnp.int32),
            pltpu.VMEM((NCH, 128), jnp.float32),
            pltpu.VMEM((128, D), jnp.float32),
            pltpu.VMEM((128, D), jnp.float32),
            pltpu.VMEM((128,), jnp.int32),
            pltpu.VMEM((128,), jnp.int32),
            pltpu.VMEM((16,), jnp.float32),
            pltpu.VMEM((HALF // 16 + 16,), jnp.float32),
            pltpu.SemaphoreType.DMA,
            pltpu.VMEM_SHARED((QROWS, D), jnp.float32),
            pltpu.VMEM_SHARED((HALF + 256,), jnp.float32),
        ],
    )


_edge_agg = _make_edge_agg()


def _make_finish(relu):
    def _finish_body(outp_ref, denp_ref, b_ref, batch_ref, h_ref, emb_ref,
                     cnt_ref):
        i = pl.program_id(0)
        den = denp_ref[...].reshape(BF) + 1e-16
        h = outp_ref[...] / den[:, None] + b_ref[...]
        if relu:
            h = jnp.maximum(h, 0.0)
        h_ref[...] = h
        bt = batch_ref[0]
        valid = i * BF + lax.broadcasted_iota(jnp.int32, (G, BF), 1) < N
        onehot = ((bt[None, :] == lax.broadcasted_iota(jnp.int32, (G, BF), 0))
                  & valid).astype(jnp.float32)
        part = jnp.dot(onehot, h, preferred_element_type=jnp.float32)
        c = jnp.broadcast_to(jnp.sum(onehot, axis=1, keepdims=True), (G, D))

        @pl.when(i == 0)
        def _():
            emb_ref[...] = jnp.zeros_like(emb_ref)
            cnt_ref[...] = jnp.zeros_like(cnt_ref)

        emb_ref[...] += part
        cnt_ref[...] += c

        @pl.when(i == GRIDF - 1)
        def _():
            emb_ref[...] = emb_ref[...] / jnp.maximum(cnt_ref[...], 1.0)

    return pl.pallas_call(
        _finish_body,
        grid=(GRIDF,),
        in_specs=[
            pl.BlockSpec((BF, D), lambda i: (i, 0)),
            pl.BlockSpec((BF // 128, 128), lambda i: (i, 0)),
            pl.BlockSpec((1, D), lambda i: (0, 0)),
            pl.BlockSpec((1, BF), lambda i: (0, i)),
        ],
        out_specs=[
            pl.BlockSpec((BF, D), lambda i: (i, 0)),
            pl.BlockSpec((G, D), lambda i: (0, 0)),
        ],
        out_shape=[
            jax.ShapeDtypeStruct((NP, D), jnp.float32),
            jax.ShapeDtypeStruct((G, D), jnp.float32),
        ],
        scratch_shapes=[pltpu.VMEM((G, D), jnp.float32)],
    )


_finish_relu = _make_finish(True)
_finish_plain = _make_finish(False)


def kernel(x, edge_index, edge_weight, batch, W1, a_src1, a_dst1, b1, W2,
           a_src2, a_dst2, b2):
    src3, dst3 = _prep(edge_index.astype(jnp.int32).reshape(2, E // 128, 128))
    batch2 = batch.astype(jnp.int32).reshape(1, N)
    b1r = b1.reshape(1, D)
    b2r = b2.reshape(1, D)

    hp1, s1, d1, m1 = _project(x, W1, a_src1, a_dst1)
    outp1, denp1 = _edge_agg(hp1, src3, dst3, s1, d1,
                             jnp.broadcast_to(m1.reshape(1), (16,)))
    h1, emb1 = _finish_relu(outp1, denp1.reshape(NP // 128, 128), b1r,
                            batch2)

    hp2, s2, d2, m2 = _project(h1, W2, a_src2, a_dst2)
    outp2, denp2 = _edge_agg(hp2, src3, dst3, s2, d2,
                             jnp.broadcast_to(m2.reshape(1), (16,)))
    _, emb2 = _finish_plain(outp2, denp2.reshape(NP // 128, 128), b2r,
                            batch2)

    return (emb1, emb2)


# speedup vs baseline: 2.0890x; 1.4512x over previous
import functools

import jax
import jax.numpy as jnp
from jax import lax
from jax.experimental import pallas as pl
from jax.experimental.pallas import tpu as pltpu
from jax.experimental.pallas import tpu_sc as plsc

N = 10000
E = 320000
D = 128
G = 64

NP = 10240
HALF = NP // 2
ET = E + N
NCH = 168
EPT = NCH * 128
ET_PAD = 16 * EPT

QROWS = HALF // 4
DTR = HALF

BF = 1024
GRIDF = NP // BF


def _proj_body(x_ref, w_ref, as_ref, ad_ref, hp_ref, s_ref, d_ref, m_ref,
               mm_ref, *, grid):
    i = pl.program_id(0)
    hp = jnp.dot(x_ref[...], w_ref[...], preferred_element_type=jnp.float32)
    hp_ref[...] = hp
    s = jnp.sum(hp * as_ref[...], axis=1)
    d = jnp.sum(hp * ad_ref[...], axis=1)
    s_ref[...] = s.reshape(BF // 128, 128)
    d_ref[...] = d.reshape(BF // 128, 128)
    valid = i * BF + lax.iota(jnp.int32, BF) < N
    ms = jnp.max(jnp.where(valid, s, -3e38))
    md = jnp.max(jnp.where(valid, d, -3e38))

    @pl.when(i == 0)
    def _():
        mm_ref[0] = ms
        mm_ref[1] = md

    @pl.when(i > 0)
    def _():
        mm_ref[0] = jnp.maximum(mm_ref[0], ms)
        mm_ref[1] = jnp.maximum(mm_ref[1], md)

    @pl.when(i == grid - 1)
    def _():
        z = mm_ref[0] + mm_ref[1]
        m_ref[0, 0] = jnp.where(z >= 0, z, 0.2 * z)


def _project(h, W, a_s, a_d):
    grid = GRIDF
    body = functools.partial(_proj_body, grid=grid)
    outs = pl.pallas_call(
        body,
        grid=(grid,),
        in_specs=[
            pl.BlockSpec((BF, D), lambda i: (i, 0)),
            pl.BlockSpec((D, D), lambda i: (0, 0)),
            pl.BlockSpec((1, D), lambda i: (0, 0)),
            pl.BlockSpec((1, D), lambda i: (0, 0)),
        ],
        out_specs=[
            pl.BlockSpec((BF, D), lambda i: (i, 0)),
            pl.BlockSpec((BF // 128, 128), lambda i: (i, 0)),
            pl.BlockSpec((BF // 128, 128), lambda i: (i, 0)),
            pl.BlockSpec((1, 1), lambda i: (0, 0), memory_space=pltpu.SMEM),
        ],
        out_shape=[
            jax.ShapeDtypeStruct((NP, D), jnp.float32),
            jax.ShapeDtypeStruct((NP // 128, 128), jnp.float32),
            jax.ShapeDtypeStruct((NP // 128, 128), jnp.float32),
            jax.ShapeDtypeStruct((1, 1), jnp.float32),
        ],
        scratch_shapes=[pltpu.SMEM((2,), jnp.float32)],
    )(h, W, a_s.reshape(1, D), a_d.reshape(1, D))
    hp, s, d, m = outs
    return hp, s, d, m


def _prep_body(e_ref, src_ref, dst_ref):
    tail_rows = 16 * NCH - E // 128
    fi = (lax.broadcasted_iota(jnp.int32, (tail_rows, 128), 0) * 128
          + lax.broadcasted_iota(jnp.int32, (tail_rows, 128), 1))
    tail = jnp.where(fi < N, fi, 0)
    for k, out_ref in ((0, src_ref), (1, dst_ref)):
        cat = jnp.concatenate([e_ref[k], tail], axis=0)
        out_ref[...] = cat.reshape(16, NCH, 128)


def _prep(e3):
    return pl.pallas_call(
        _prep_body,
        out_shape=[
            jax.ShapeDtypeStruct((16, NCH, 128), jnp.int32),
            jax.ShapeDtypeStruct((16, NCH, 128), jnp.int32),
        ],
    )(e3)


def _edge_body(hp_hbm, src_hbm, dst_hbm, asrc_hbm, adst_hbm, m_hbm,
               outp_hbm, denp_hbm,
               asrc_v, adst_v, src_v, dst_v, ex_v, rows_g, rows_s, idq_v,
               idd_v, wbuf_v, m_v, zden_v, sem,
               out_sp, den_sp):
    cid = lax.axis_index("c")
    sid = lax.axis_index("s")
    gbase = cid * HALF

    pltpu.sync_copy(asrc_hbm, asrc_v)
    pltpu.sync_copy(adst_hbm, adst_v)
    pltpu.sync_copy(src_hbm.at[sid], src_v)
    pltpu.sync_copy(dst_hbm.at[sid], dst_v)
    pltpu.sync_copy(m_hbm, m_v)
    mvec = m_v[...]
    zcol = jnp.zeros((16,), jnp.int32)

    zero16 = jnp.zeros((16,), jnp.float32)

    def _zr(r, carry):
        for j in range(D // 16):
            rows_g[r, pl.ds(j * 16, 16)] = zero16
        return carry

    def _zd(k, carry):
        zden_v[pl.ds(k * 16, 16)] = zero16
        return carry

    lax.fori_loop(0, (HALF // 16 + 16) // 16, _zd, 0)

    pltpu.sync_copy(zden_v, den_sp.at[pl.ds(sid * (HALF // 16 + 16),
                                            HALF // 16 + 16)])

    plsc.subcore_barrier()

    ebase = sid * EPT

    def _phase_a(c, carry):
        for j in range(8):
            sl = pl.ds(j * 16, 16)
            si = src_v[c, sl]
            di = dst_v[c, sl]
            s = plsc.load_gather(asrc_v, [lax.shift_right_logical(si, 7),
                                          lax.bitwise_and(si, 127)])
            d = plsc.load_gather(adst_v, [lax.shift_right_logical(di, 7),
                                          lax.bitwise_and(di, 127)])
            z = s + d
            e = jnp.where(z >= 0, z, 0.2 * z)
            ex = jnp.exp(e - mvec)
            gi = ebase + c * 128 + j * 16 + lax.iota(jnp.int32, 16)
            ex = jnp.where(gi < ET, ex, 0.0)
            ex_v[c, sl] = ex
            ld = di - gbase
            inh = (ld >= 0) & (ld < HALF)
            idd_v[sl] = jnp.where(inh, ld, DTR)
        pltpu.sync_copy(ex_v.at[c], den_sp.at[idd_v], add=True)
        return carry

    lax.fori_loop(0, NCH, _phase_a, 0)

    spt = QROWS // 16

    for q in range(HALF // QROWS):
        qb = q * QROWS

        lax.fori_loop(0, spt, _zr, 0)
        pltpu.sync_copy(rows_g.at[pl.ds(0, spt)],
                        out_sp.at[pl.ds(sid * spt, spt)])
        plsc.subcore_barrier()

        def _process(c, gbuf):
            for j in range(8):
                sl = pl.ds(j * 16, 16)
                ld = dst_v[c, sl] - gbase
                idq_v[sl] = jnp.clip(ld - qb, 0, QROWS - 1)
                wv = ex_v[c, sl]
                wbuf_v[sl] = jnp.where((ld >= qb) & (ld < qb + QROWS), wv,
                                       0.0)

            def _scale(r4, carry2):
                for u in range(4):
                    r = r4 * 4 + u
                    w = plsc.load_gather(wbuf_v,
                                         [jnp.full((16,), r, jnp.int32)])
                    for j in range(D // 16):
                        sl = pl.ds(j * 16, 16)
                        gbuf[r, sl] = gbuf[r, sl] * w
                return carry2

            lax.fori_loop(0, 32, _scale, 0)
            pltpu.sync_copy(gbuf, out_sp.at[idq_v], add=True)

        pltpu.async_copy(hp_hbm.at[src_v.at[0]], rows_g, sem).wait()

        def _phase_b(k, carry):
            c0 = 2 * k
            c1 = 2 * k + 1
            pltpu.async_copy(hp_hbm.at[src_v.at[c1]], rows_s, sem)
            _process(c0, rows_g)
            pltpu.make_async_copy(hp_hbm.at[src_v.at[c1]], rows_s, sem).wait()

            @pl.when(c1 + 1 < NCH)
            def _():
                pltpu.async_copy(hp_hbm.at[src_v.at[c1 + 1]], rows_g, sem)
            _process(c1, rows_s)

            @pl.when(c1 + 1 < NCH)
            def _():
                pltpu.make_async_copy(hp_hbm.at[src_v.at[c1 + 1]], rows_g,
                                      sem).wait()
            return carry

        lax.fori_loop(0, NCH // 2, _phase_b, 0)
        plsc.subcore_barrier()

        pltpu.sync_copy(out_sp.at[pl.ds(sid * spt, spt)],
                        outp_hbm.at[pl.ds(gbase + qb + sid * spt, spt)])
        plsc.subcore_barrier()

    dpt = HALF // 16
    pltpu.sync_copy(den_sp.at[pl.ds(sid * dpt, dpt)],
                    denp_hbm.at[pl.ds(gbase + sid * dpt, dpt)])


def _make_edge_agg(interpret=False):
    return pl.kernel(
        _edge_body,
        out_type=[
            jax.ShapeDtypeStruct((NP, D), jnp.float32),
            jax.ShapeDtypeStruct((NP,), jnp.float32),
        ],
        mesh=plsc.VectorSubcoreMesh(core_axis_name="c", subcore_axis_name="s"),
        compiler_params=pltpu.CompilerParams(needs_layout_passes=False,
                                             use_tc_tiling_on_sc=False),
        interpret=interpret,
        scratch_types=[
            pltpu.VMEM((NP // 128, 128), jnp.float32),
            pltpu.VMEM((NP // 128, 128), jnp.float32),
            pltpu.VMEM((NCH, 128), jnp.int32),
            pltpu.VMEM((NCH, 128), jnp.int32),
            pltpu.VMEM((NCH, 128), jnp.float32),
            pltpu.VMEM((128, D), jnp.float32),
            pltpu.VMEM((128, D), jnp.float32),
            pltpu.VMEM((128,), jnp.int32),
            pltpu.VMEM((128,), jnp.int32),
            pltpu.VMEM((128,), jnp.float32),
            pltpu.VMEM((16,), jnp.float32),
            pltpu.VMEM((HALF // 16 + 16,), jnp.float32),
            pltpu.SemaphoreType.DMA,
            pltpu.VMEM_SHARED((QROWS, D), jnp.float32),
            pltpu.VMEM_SHARED((HALF + 256,), jnp.float32),
        ],
    )


_edge_agg = _make_edge_agg()


def _make_finish(relu):
    def _finish_body(outp_ref, denp_ref, b_ref, batch_ref, h_ref, emb_ref,
                     cnt_ref):
        i = pl.program_id(0)
        den = denp_ref[...].reshape(BF) + 1e-16
        h = outp_ref[...] / den[:, None] + b_ref[...]
        if relu:
            h = jnp.maximum(h, 0.0)
        h_ref[...] = h
        bt = batch_ref[0]
        valid = i * BF + lax.broadcasted_iota(jnp.int32, (G, BF), 1) < N
        onehot = ((bt[None, :] == lax.broadcasted_iota(jnp.int32, (G, BF), 0))
                  & valid).astype(jnp.float32)
        part = jnp.dot(onehot, h, preferred_element_type=jnp.float32)
        c = jnp.broadcast_to(jnp.sum(onehot, axis=1, keepdims=True), (G, D))

        @pl.when(i == 0)
        def _():
            emb_ref[...] = jnp.zeros_like(emb_ref)
            cnt_ref[...] = jnp.zeros_like(cnt_ref)

        emb_ref[...] += part
        cnt_ref[...] += c

        @pl.when(i == GRIDF - 1)
        def _():
            emb_ref[...] = emb_ref[...] / jnp.maximum(cnt_ref[...], 1.0)

    return pl.pallas_call(
        _finish_body,
        grid=(GRIDF,),
        in_specs=[
            pl.BlockSpec((BF, D), lambda i: (i, 0)),
            pl.BlockSpec((BF // 128, 128), lambda i: (i, 0)),
            pl.BlockSpec((1, D), lambda i: (0, 0)),
            pl.BlockSpec((1, BF), lambda i: (0, i)),
        ],
        out_specs=[
            pl.BlockSpec((BF, D), lambda i: (i, 0)),
            pl.BlockSpec((G, D), lambda i: (0, 0)),
        ],
        out_shape=[
            jax.ShapeDtypeStruct((NP, D), jnp.float32),
            jax.ShapeDtypeStruct((G, D), jnp.float32),
        ],
        scratch_shapes=[pltpu.VMEM((G, D), jnp.float32)],
    )


_finish_relu = _make_finish(True)
_finish_plain = _make_finish(False)


def kernel(x, edge_index, edge_weight, batch, W1, a_src1, a_dst1, b1, W2,
           a_src2, a_dst2, b2):
    src3, dst3 = _prep(edge_index.astype(jnp.int32).reshape(2, E // 128, 128))
    batch2 = batch.astype(jnp.int32).reshape(1, N)
    b1r = b1.reshape(1, D)
    b2r = b2.reshape(1, D)

    hp1, s1, d1, m1 = _project(x, W1, a_src1, a_dst1)
    outp1, denp1 = _edge_agg(hp1, src3, dst3, s1, d1,
                             jnp.broadcast_to(m1.reshape(1), (16,)))
    h1, emb1 = _finish_relu(outp1, denp1.reshape(NP // 128, 128), b1r,
                            batch2)

    hp2, s2, d2, m2 = _project(h1, W2, a_src2, a_dst2)
    outp2, denp2 = _edge_agg(hp2, src3, dst3, s2, d2,
                             jnp.broadcast_to(m2.reshape(1), (16,)))
    _, emb2 = _finish_plain(outp2, denp2.reshape(NP // 128, 128), b2r,
                            batch2)

    return (emb1, emb2)


# speedup vs baseline: 2.0896x; 1.0003x over previous
import functools

import jax
import jax.numpy as jnp
from jax import lax
from jax.experimental import pallas as pl
from jax.experimental.pallas import tpu as pltpu
from jax.experimental.pallas import tpu_sc as plsc

N = 10000
E = 320000
D = 128
G = 64

NP = 10240
HALF = NP // 2
ET = E + N
NCH = 168
EPT = NCH * 128
ET_PAD = 16 * EPT

QROWS = HALF // 4
DTR = HALF

BF = 1024
GRIDF = NP // BF


def _proj_body(x_ref, w_ref, as_ref, ad_ref, hp_ref, s_ref, d_ref, m_ref,
               mm_ref, *, grid):
    i = pl.program_id(0)
    hp = jnp.dot(x_ref[...], w_ref[...], preferred_element_type=jnp.float32)
    hp_ref[...] = hp
    s = jnp.sum(hp * as_ref[...], axis=1)
    d = jnp.sum(hp * ad_ref[...], axis=1)
    s_ref[...] = s.reshape(BF // 128, 128)
    d_ref[...] = d.reshape(BF // 128, 128)
    valid = i * BF + lax.iota(jnp.int32, BF) < N
    ms = jnp.max(jnp.where(valid, s, -3e38))
    md = jnp.max(jnp.where(valid, d, -3e38))

    @pl.when(i == 0)
    def _():
        mm_ref[0] = ms
        mm_ref[1] = md

    @pl.when(i > 0)
    def _():
        mm_ref[0] = jnp.maximum(mm_ref[0], ms)
        mm_ref[1] = jnp.maximum(mm_ref[1], md)

    @pl.when(i == grid - 1)
    def _():
        z = mm_ref[0] + mm_ref[1]
        m_ref[0, 0] = jnp.where(z >= 0, z, 0.2 * z)


def _project(h, W, a_s, a_d):
    grid = GRIDF
    body = functools.partial(_proj_body, grid=grid)
    outs = pl.pallas_call(
        body,
        grid=(grid,),
        in_specs=[
            pl.BlockSpec((BF, D), lambda i: (i, 0)),
            pl.BlockSpec((D, D), lambda i: (0, 0)),
            pl.BlockSpec((1, D), lambda i: (0, 0)),
            pl.BlockSpec((1, D), lambda i: (0, 0)),
        ],
        out_specs=[
            pl.BlockSpec((BF, D), lambda i: (i, 0)),
            pl.BlockSpec((BF // 128, 128), lambda i: (i, 0)),
            pl.BlockSpec((BF // 128, 128), lambda i: (i, 0)),
            pl.BlockSpec((1, 1), lambda i: (0, 0), memory_space=pltpu.SMEM),
        ],
        out_shape=[
            jax.ShapeDtypeStruct((NP, D), jnp.float32),
            jax.ShapeDtypeStruct((NP // 128, 128), jnp.float32),
            jax.ShapeDtypeStruct((NP // 128, 128), jnp.float32),
            jax.ShapeDtypeStruct((1, 1), jnp.float32),
        ],
        scratch_shapes=[pltpu.SMEM((2,), jnp.float32)],
    )(h, W, a_s.reshape(1, D), a_d.reshape(1, D))
    hp, s, d, m = outs
    return hp, s, d, m


def _prep_body(e_ref, src_ref, dst_ref):
    tail_rows = 16 * NCH - E // 128
    fi = (lax.broadcasted_iota(jnp.int32, (tail_rows, 128), 0) * 128
          + lax.broadcasted_iota(jnp.int32, (tail_rows, 128), 1))
    tail = jnp.where(fi < N, fi, 0)
    for k, out_ref in ((0, src_ref), (1, dst_ref)):
        cat = jnp.concatenate([e_ref[k], tail], axis=0)
        out_ref[...] = cat.reshape(16, NCH, 128)


def _prep(e3):
    return pl.pallas_call(
        _prep_body,
        out_shape=[
            jax.ShapeDtypeStruct((16, NCH, 128), jnp.int32),
            jax.ShapeDtypeStruct((16, NCH, 128), jnp.int32),
        ],
    )(e3)


def _edge_body(hp_hbm, src_hbm, dst_hbm, asrc_hbm, adst_hbm, m_hbm,
               outp_hbm, denp_hbm,
               asrc_v, adst_v, src_v, dst_v, ex_v, rows_g, rows_s, idq_v,
               idd_v, wbuf_v, m_v, zden_v, sem,
               out_sp, den_sp):
    cid = lax.axis_index("c")
    sid = lax.axis_index("s")
    gbase = cid * HALF

    pltpu.sync_copy(asrc_hbm, asrc_v)
    pltpu.sync_copy(adst_hbm, adst_v)
    pltpu.sync_copy(src_hbm.at[sid], src_v)
    pltpu.sync_copy(dst_hbm.at[sid], dst_v)
    pltpu.sync_copy(m_hbm, m_v)
    mvec = m_v[...]

    zero16 = jnp.zeros((16,), jnp.float32)

    def _zr(r, carry):
        for j in range(D // 16):
            rows_g[r, pl.ds(j * 16, 16)] = zero16
        return carry

    def _zd(k, carry):
        zden_v[pl.ds(k * 16, 16)] = zero16
        return carry

    lax.fori_loop(0, (HALF // 16 + 16) // 16, _zd, 0)

    pltpu.sync_copy(zden_v, den_sp.at[pl.ds(sid * (HALF // 16 + 16),
                                            HALF // 16 + 16)])

    plsc.subcore_barrier()

    ebase = sid * EPT

    def _phase_a(c, carry):
        for j in range(8):
            sl = pl.ds(j * 16, 16)
            si = src_v[c, sl]
            di = dst_v[c, sl]
            s = plsc.load_gather(asrc_v, [lax.shift_right_logical(si, 7),
                                          lax.bitwise_and(si, 127)])
            d = plsc.load_gather(adst_v, [lax.shift_right_logical(di, 7),
                                          lax.bitwise_and(di, 127)])
            z = s + d
            e = jnp.where(z >= 0, z, 0.2 * z)
            ex = jnp.exp(e - mvec)
            gi = ebase + c * 128 + j * 16 + lax.iota(jnp.int32, 16)
            ex = jnp.where(gi < ET, ex, 0.0)
            ex_v[c, sl] = ex
            ld = di - gbase
            inh = (ld >= 0) & (ld < HALF)
            idd_v[sl] = jnp.where(inh, ld, DTR)
        pltpu.sync_copy(ex_v.at[c], den_sp.at[idd_v], add=True)
        return carry

    lax.fori_loop(0, NCH, _phase_a, 0)

    spt = QROWS // 16

    for q in range(HALF // QROWS):
        qb = q * QROWS

        lax.fori_loop(0, spt, _zr, 0)
        pltpu.sync_copy(rows_g.at[pl.ds(0, spt)],
                        out_sp.at[pl.ds(sid * spt, spt)])
        plsc.subcore_barrier()

        def _process(c, gbuf):
            for j in range(8):
                sl = pl.ds(j * 16, 16)
                ld = dst_v[c, sl] - gbase
                idq_v[sl] = jnp.clip(ld - qb, 0, QROWS - 1)
                wv = ex_v[c, sl]
                wbuf_v[sl] = jnp.where((ld >= qb) & (ld < qb + QROWS), wv,
                                       0.0)

            def _scale(r4, carry2):
                for u in range(4):
                    r = r4 * 4 + u
                    w = plsc.load_gather(wbuf_v,
                                         [jnp.full((16,), r, jnp.int32)])
                    for j in range(D // 16):
                        sl = pl.ds(j * 16, 16)
                        gbuf[r, sl] = gbuf[r, sl] * w
                return carry2

            lax.fori_loop(0, 32, _scale, 0)
            pltpu.sync_copy(gbuf, out_sp.at[idq_v], add=True)

        pltpu.async_copy(hp_hbm.at[src_v.at[0]], rows_g, sem).wait()

        def _phase_b(k, carry):
            c0 = 2 * k
            c1 = 2 * k + 1
            pltpu.async_copy(hp_hbm.at[src_v.at[c1]], rows_s, sem)
            _process(c0, rows_g)
            pltpu.make_async_copy(hp_hbm.at[src_v.at[c1]], rows_s, sem).wait()

            @pl.when(c1 + 1 < NCH)
            def _():
                pltpu.async_copy(hp_hbm.at[src_v.at[c1 + 1]], rows_g, sem)
            _process(c1, rows_s)

            @pl.when(c1 + 1 < NCH)
            def _():
                pltpu.make_async_copy(hp_hbm.at[src_v.at[c1 + 1]], rows_g,
                                      sem).wait()
            return carry

        lax.fori_loop(0, NCH // 2, _phase_b, 0)
        plsc.subcore_barrier()

        pltpu.sync_copy(out_sp.at[pl.ds(sid * spt, spt)],
                        outp_hbm.at[pl.ds(gbase + qb + sid * spt, spt)])
        plsc.subcore_barrier()

    dpt = HALF // 16
    pltpu.sync_copy(den_sp.at[pl.ds(sid * dpt, dpt)],
                    denp_hbm.at[pl.ds(gbase + sid * dpt, dpt)])


def _make_edge_agg(interpret=False):
    return pl.kernel(
        _edge_body,
        out_type=[
            jax.ShapeDtypeStruct((NP, D), jnp.float32),
            jax.ShapeDtypeStruct((NP,), jnp.float32),
        ],
        mesh=plsc.VectorSubcoreMesh(core_axis_name="c", subcore_axis_name="s"),
        compiler_params=pltpu.CompilerParams(needs_layout_passes=False,
                                             use_tc_tiling_on_sc=False),
        interpret=interpret,
        scratch_types=[
            pltpu.VMEM((NP // 128, 128), jnp.float32),
            pltpu.VMEM((NP // 128, 128), jnp.float32),
            pltpu.VMEM((NCH, 128), jnp.int32),
            pltpu.VMEM((NCH, 128), jnp.int32),
            pltpu.VMEM((NCH, 128), jnp.float32),
            pltpu.VMEM((128, D), jnp.float32),
            pltpu.VMEM((128, D), jnp.float32),
            pltpu.VMEM((128,), jnp.int32),
            pltpu.VMEM((128,), jnp.int32),
            pltpu.VMEM((128,), jnp.float32),
            pltpu.VMEM((16,), jnp.float32),
            pltpu.VMEM((HALF // 16 + 16,), jnp.float32),
            pltpu.SemaphoreType.DMA,
            pltpu.VMEM_SHARED((QROWS, D), jnp.float32),
            pltpu.VMEM_SHARED((HALF + 256,), jnp.float32),
        ],
    )


_edge_agg = _make_edge_agg()


def _make_finish(relu):
    def _finish_body(outp_ref, denp_ref, b_ref, batch_ref, h_ref, emb_ref,
                     cnt_ref):
        i = pl.program_id(0)
        den = denp_ref[...].reshape(BF) + 1e-16
        h = outp_ref[...] / den[:, None] + b_ref[...]
        if relu:
            h = jnp.maximum(h, 0.0)
        h_ref[...] = h
        bt = batch_ref[0]
        valid = i * BF + lax.broadcasted_iota(jnp.int32, (G, BF), 1) < N
        onehot = ((bt[None, :] == lax.broadcasted_iota(jnp.int32, (G, BF), 0))
                  & valid).astype(jnp.float32)
        part = jnp.dot(onehot, h, preferred_element_type=jnp.float32)
        c = jnp.broadcast_to(jnp.sum(onehot, axis=1, keepdims=True), (G, D))

        @pl.when(i == 0)
        def _():
            emb_ref[...] = jnp.zeros_like(emb_ref)
            cnt_ref[...] = jnp.zeros_like(cnt_ref)

        emb_ref[...] += part
        cnt_ref[...] += c

        @pl.when(i == GRIDF - 1)
        def _():
            emb_ref[...] = emb_ref[...] / jnp.maximum(cnt_ref[...], 1.0)

    return pl.pallas_call(
        _finish_body,
        grid=(GRIDF,),
        in_specs=[
            pl.BlockSpec((BF, D), lambda i: (i, 0)),
            pl.BlockSpec((BF // 128, 128), lambda i: (i, 0)),
            pl.BlockSpec((1, D), lambda i: (0, 0)),
            pl.BlockSpec((1, BF), lambda i: (0, i)),
        ],
        out_specs=[
            pl.BlockSpec((BF, D), lambda i: (i, 0)),
            pl.BlockSpec((G, D), lambda i: (0, 0)),
        ],
        out_shape=[
            jax.ShapeDtypeStruct((NP, D), jnp.float32),
            jax.ShapeDtypeStruct((G, D), jnp.float32),
        ],
        scratch_shapes=[pltpu.VMEM((G, D), jnp.float32)],
    )


_finish_relu = _make_finish(True)
_finish_plain = _make_finish(False)


def kernel(x, edge_index, edge_weight, batch, W1, a_src1, a_dst1, b1, W2,
           a_src2, a_dst2, b2):
    src3, dst3 = _prep(edge_index.astype(jnp.int32).reshape(2, E // 128, 128))
    batch2 = batch.astype(jnp.int32).reshape(1, N)
    b1r = b1.reshape(1, D)
    b2r = b2.reshape(1, D)

    hp1, s1, d1, m1 = _project(x, W1, a_src1, a_dst1)
    outp1, denp1 = _edge_agg(hp1, src3, dst3, s1, d1,
                             jnp.broadcast_to(m1.reshape(1), (16,)))
    h1, emb1 = _finish_relu(outp1, denp1.reshape(NP // 128, 128), b1r,
                            batch2)

    hp2, s2, d2, m2 = _project(h1, W2, a_src2, a_dst2)
    outp2, denp2 = _edge_agg(hp2, src3, dst3, s2, d2,
                             jnp.broadcast_to(m2.reshape(1), (16,)))
    _, emb2 = _finish_plain(outp2, denp2.reshape(NP // 128, 128), b2r,
                            batch2)

    return (emb1, emb2)
